# baseline (speedup 1.0000x reference)
import functools

import jax
import jax.numpy as jnp
from jax import lax
from jax.experimental import pallas as pl
from jax.experimental.pallas import tpu as pltpu
from jax.experimental.pallas import tpu_sc as plsc

VOCAB = 1000
N_EMBED = 128
T = 50
B = 1024
NTOK = B * T

_info = plsc.get_sparse_core_info()
NC, NS, L = _info.num_cores, _info.num_subcores, _info.num_lanes
NW = NC * NS
BPW = NTOK // NW
CH = 80
NCH = BPW // CH

_sc_mesh = plsc.VectorSubcoreMesh(core_axis_name="c", subcore_axis_name="s")


@functools.partial(
    pl.kernel,
    mesh=_sc_mesh,
    out_type=jax.ShapeDtypeStruct((NTOK, N_EMBED), jnp.float32),
    scratch_types=[
        pltpu.VMEM((BPW,), jnp.int32),
        pltpu.VMEM((CH, N_EMBED), jnp.float32),
        pltpu.VMEM((CH, N_EMBED), jnp.float32),
        pltpu.SemaphoreType.DMA,
        pltpu.SemaphoreType.DMA,
    ],
)
def _sc_gather(idx_hbm, table_hbm, out_hbm, idx_v, rows_a, rows_b, sem_a, sem_b):
    wid = lax.axis_index("s") * NC + lax.axis_index("c")
    base = wid * BPW
    pltpu.sync_copy(idx_hbm.at[pl.ds(base, BPW)], idx_v)

    def two_chunks(j, carry):
        off_a = base + (2 * j) * CH
        off_b = base + (2 * j + 1) * CH
        ga = pltpu.async_copy(
            table_hbm.at[idx_v.at[pl.ds((2 * j) * CH, CH)]], rows_a, sem_a)
        gb = pltpu.async_copy(
            table_hbm.at[idx_v.at[pl.ds((2 * j + 1) * CH, CH)]], rows_b, sem_b)
        ga.wait()
        pltpu.sync_copy(rows_a, out_hbm.at[pl.ds(off_a, CH)])
        gb.wait()
        pltpu.sync_copy(rows_b, out_hbm.at[pl.ds(off_b, CH)])
        return carry

    lax.fori_loop(0, NCH // 2, two_chunks, 0)


BB = 8
TOK = BB * T
GRID = B // BB


def _mm_body(x_ref, pos_ref, w_ref, b_ref, out_ref):
    x = x_ref[...] + pos_ref[...]
    out_ref[...] = (
        jnp.dot(x, w_ref[...], preferred_element_type=jnp.float32) + b_ref[...]
    )


def kernel(idx, tok_table, pos_table, W, b):
    idx_flat = idx.astype(jnp.int32).reshape(NTOK)
    tok_emb = _sc_gather(idx_flat, tok_table)

    pos_tiled = jnp.tile(pos_table, (BB, 1))
    b2 = b.reshape(1, VOCAB)
    out = pl.pallas_call(
        _mm_body,
        grid=(GRID,),
        in_specs=[
            pl.BlockSpec((TOK, N_EMBED), lambda i: (i, 0)),
            pl.BlockSpec((TOK, N_EMBED), lambda i: (0, 0)),
            pl.BlockSpec((N_EMBED, VOCAB), lambda i: (0, 0)),
            pl.BlockSpec((1, VOCAB), lambda i: (0, 0)),
        ],
        out_specs=pl.BlockSpec((TOK, VOCAB), lambda i: (i, 0)),
        out_shape=jax.ShapeDtypeStruct((NTOK, VOCAB), jnp.float32),
    )(tok_emb, pos_tiled, W, b2)
    return out.reshape(B, T, VOCAB)


# speedup vs baseline: 1.2427x; 1.2427x over previous
import functools

import jax
import jax.numpy as jnp
from jax import lax
from jax.experimental import pallas as pl
from jax.experimental.pallas import tpu as pltpu
from jax.experimental.pallas import tpu_sc as plsc

VOCAB = 1000
N_EMBED = 128
T = 50
B = 1024
NTOK = B * T

_info = plsc.get_sparse_core_info()
NC, NS, L = _info.num_cores, _info.num_subcores, _info.num_lanes
NW = NC * NS
BPW = NTOK // NW
CH = 80
NCH = BPW // CH

_sc_mesh = plsc.VectorSubcoreMesh(core_axis_name="c", subcore_axis_name="s")


@functools.partial(
    pl.kernel,
    mesh=_sc_mesh,
    out_type=jax.ShapeDtypeStruct((NTOK, N_EMBED), jnp.float32),
    scratch_types=[
        pltpu.VMEM((BPW,), jnp.int32),
        pltpu.VMEM((CH, N_EMBED), jnp.float32),
        pltpu.VMEM((CH, N_EMBED), jnp.float32),
        pltpu.SemaphoreType.DMA,
        pltpu.SemaphoreType.DMA,
    ],
)
def _sc_gather(idx_hbm, table_hbm, out_hbm, idx_v, rows_a, rows_b, sem_a, sem_b):
    wid = lax.axis_index("s") * NC + lax.axis_index("c")
    base = wid * BPW
    pltpu.sync_copy(idx_hbm.at[pl.ds(base, BPW)], idx_v)

    def two_chunks(j, carry):
        off_a = base + (2 * j) * CH
        off_b = base + (2 * j + 1) * CH
        ga = pltpu.async_copy(
            table_hbm.at[idx_v.at[pl.ds((2 * j) * CH, CH)]], rows_a, sem_a)
        gb = pltpu.async_copy(
            table_hbm.at[idx_v.at[pl.ds((2 * j + 1) * CH, CH)]], rows_b, sem_b)
        ga.wait()
        pltpu.sync_copy(rows_a, out_hbm.at[pl.ds(off_a, CH)])
        gb.wait()
        pltpu.sync_copy(rows_b, out_hbm.at[pl.ds(off_b, CH)])
        return carry

    lax.fori_loop(0, NCH // 2, two_chunks, 0)


BB = 8
TOK = BB * T
GRID = B // BB


def _mm_body(x_ref, pos_ref, w_ref, b_ref, out_ref):
    x = x_ref[...] + pos_ref[...]
    y = jnp.dot(x, w_ref[...], preferred_element_type=jnp.float32) + b_ref[...]
    for r in range(BB):
        out_ref[r, :, :] = y[r * T:(r + 1) * T, :]


def kernel(idx, tok_table, pos_table, W, b):
    idx_flat = idx.astype(jnp.int32).reshape(NTOK)
    tok_emb = _sc_gather(idx_flat, tok_table)

    pos_tiled = jnp.tile(pos_table, (BB, 1))
    b2 = b.reshape(1, VOCAB)
    out = pl.pallas_call(
        _mm_body,
        grid=(GRID,),
        in_specs=[
            pl.BlockSpec((TOK, N_EMBED), lambda i: (i, 0)),
            pl.BlockSpec((TOK, N_EMBED), lambda i: (0, 0)),
            pl.BlockSpec((N_EMBED, VOCAB), lambda i: (0, 0)),
            pl.BlockSpec((1, VOCAB), lambda i: (0, 0)),
        ],
        out_specs=pl.BlockSpec((BB, T, VOCAB), lambda i: (i, 0, 0)),
        out_shape=jax.ShapeDtypeStruct((B, T, VOCAB), jnp.float32),
    )(tok_emb, pos_tiled, W, b2)
    return out


# speedup vs baseline: 1.4180x; 1.1411x over previous
import functools

import jax
import jax.numpy as jnp
from jax import lax
from jax.experimental import pallas as pl
from jax.experimental.pallas import tpu as pltpu
from jax.experimental.pallas import tpu_sc as plsc

VOCAB = 1000
N_EMBED = 128
T = 50
B = 1024
NTOK = B * T

_info = plsc.get_sparse_core_info()
NC, NS, L = _info.num_cores, _info.num_subcores, _info.num_lanes
NW = NC * NS
BPW = NTOK // NW
CH = 80
NCH = BPW // CH

_sc_mesh = plsc.VectorSubcoreMesh(core_axis_name="c", subcore_axis_name="s")


@functools.partial(
    pl.kernel,
    mesh=_sc_mesh,
    out_type=jax.ShapeDtypeStruct((NTOK, N_EMBED), jnp.float32),
    scratch_types=[
        pltpu.VMEM((BPW,), jnp.int32),
        pltpu.VMEM((CH, N_EMBED), jnp.float32),
        pltpu.VMEM((CH, N_EMBED), jnp.float32),
        pltpu.SemaphoreType.DMA,
        pltpu.SemaphoreType.DMA,
    ],
)
def _sc_gather(idx_hbm, table_hbm, out_hbm, idx_v, rows_a, rows_b, sem_a, sem_b):
    wid = lax.axis_index("s") * NC + lax.axis_index("c")
    base = wid * BPW
    pltpu.sync_copy(idx_hbm.at[pl.ds(base, BPW)], idx_v)

    def two_chunks(j, carry):
        off_a = base + (2 * j) * CH
        off_b = base + (2 * j + 1) * CH
        ga = pltpu.async_copy(
            table_hbm.at[idx_v.at[pl.ds((2 * j) * CH, CH)]], rows_a, sem_a)
        gb = pltpu.async_copy(
            table_hbm.at[idx_v.at[pl.ds((2 * j + 1) * CH, CH)]], rows_b, sem_b)
        ga.wait()
        pltpu.sync_copy(rows_a, out_hbm.at[pl.ds(off_a, CH)])
        gb.wait()
        pltpu.sync_copy(rows_b, out_hbm.at[pl.ds(off_b, CH)])
        return carry

    lax.fori_loop(0, NCH // 2, two_chunks, 0)


BB = 32
TOK = BB * T
GRID = B // BB


def _mm_body(x_ref, pos_ref, w_ref, b_ref, out_ref):
    x = x_ref[...] + pos_ref[...]
    y = jnp.dot(x, w_ref[...], preferred_element_type=jnp.float32) + b_ref[...]
    for r in range(BB):
        out_ref[r, :, :] = y[r * T:(r + 1) * T, :]


def kernel(idx, tok_table, pos_table, W, b):
    idx_flat = idx.astype(jnp.int32).reshape(NTOK)
    tok_emb = _sc_gather(idx_flat, tok_table)

    pos_tiled = jnp.tile(pos_table, (BB, 1))
    b2 = b.reshape(1, VOCAB)
    out = pl.pallas_call(
        _mm_body,
        grid=(GRID,),
        in_specs=[
            pl.BlockSpec((TOK, N_EMBED), lambda i: (i, 0)),
            pl.BlockSpec((TOK, N_EMBED), lambda i: (0, 0)),
            pl.BlockSpec((N_EMBED, VOCAB), lambda i: (0, 0)),
            pl.BlockSpec((1, VOCAB), lambda i: (0, 0)),
        ],
        out_specs=pl.BlockSpec((BB, T, VOCAB), lambda i: (i, 0, 0)),
        out_shape=jax.ShapeDtypeStruct((B, T, VOCAB), jnp.float32),
    )(tok_emb, pos_tiled, W, b2)
    return out
